# baseline (speedup 1.0000x reference)
import jax
import jax.numpy as jnp
from jax import lax
from jax.experimental import pallas as pl
from jax.experimental.pallas import tpu as pltpu
from jax.experimental.pallas import tpu_sc as plsc

N_POINTS = 819200
N_GRID = 512
N_BASIS = 16

NC = 2
NS = 16
NW = NC * NS

PER_W = N_POINTS // NW
CHUNK = 512
N_CHUNKS = PER_W // CHUNK
N_PAIRS = N_CHUNKS // 2


def _sc_body(t_hbm, grid_hbm, out_hbm, table_v, dtab_v,
             t_v0, t_v1, out_v0, out_v1, sem_t0, sem_t1, sem_o0, sem_o1):
    wid = lax.axis_index("s") * NC + lax.axis_index("c")
    my_base = wid * PER_W

    pltpu.sync_copy(grid_hbm, table_v)
    pltpu.sync_copy(grid_hbm.at[pl.ds(N_BASIS, (N_GRID - 1) * N_BASIS)],
                    dtab_v.at[pl.ds(0, (N_GRID - 1) * N_BASIS)])
    pltpu.sync_copy(grid_hbm.at[pl.ds((N_GRID - 1) * N_BASIS, N_BASIS)],
                    dtab_v.at[pl.ds((N_GRID - 1) * N_BASIS, N_BASIS)])

    lane = lax.iota(jnp.int32, 16)

    def diff_body(r, _):
        off = r * 16
        dtab_v[pl.ds(off, 16)] = (dtab_v[pl.ds(off, 16)]
                                  - table_v[pl.ds(off, 16)])
        return 0

    lax.fori_loop(0, N_GRID, diff_body, 0)

    def prep(t_v, j):
        tvec = t_v[pl.ds(j * 16, 16)]
        u = tvec * jnp.float32(N_GRID - 1)
        i0v = u.astype(jnp.int32)
        wv = u - i0v.astype(jnp.float32)
        return i0v * N_BASIS, wv

    def compute_chunk(t_v, out_v):
        def block_body(j, carry):
            o0v, wv = carry
            carry_n = prep(t_v, j + 1)
            rowv = j * 16 + lane
            for wave in range(0, 8, 2):
                gs = []
                for d in range(wave, wave + 2):
                    dvec = (lane + d) & (N_BASIS - 1)
                    dvec_b = dvec ^ 8
                    idx_a = o0v | dvec
                    idx_b = idx_a ^ 8
                    g0a = plsc.load_gather(table_v, [idx_a])
                    gda = plsc.load_gather(dtab_v, [idx_a])
                    g0b = plsc.load_gather(table_v, [idx_b])
                    gdb = plsc.load_gather(dtab_v, [idx_b])
                    gs.append((dvec, dvec_b, g0a, gda, g0b, gdb))
                for dvec, dvec_b, g0a, gda, g0b, gdb in gs:
                    plsc.store_scatter(out_v, [dvec, rowv], g0a + wv * gda)
                    plsc.store_scatter(out_v, [dvec_b, rowv], g0b + wv * gdb)
            return carry_n

        lax.fori_loop(0, CHUNK // 16, block_body, prep(t_v, 0))

    def t_copy(base, t_v, sem):
        return pltpu.make_async_copy(t_hbm.at[pl.ds(base, CHUNK)],
                                     t_v.at[pl.ds(0, CHUNK)], sem)

    def out_copy(base, out_v, sem):
        return pltpu.make_async_copy(out_v, out_hbm.at[:, pl.ds(base, CHUNK)],
                                     sem)

    t_v0[pl.ds(CHUNK, 16)] = jnp.zeros(16, jnp.float32)
    t_v1[pl.ds(CHUNK, 16)] = jnp.zeros(16, jnp.float32)

    t_copy(my_base, t_v0, sem_t0).start()

    def pair_body(k, _):
        b0 = my_base + (2 * k) * CHUNK
        b1 = b0 + CHUNK

        t_copy(b0, t_v0, sem_t0).wait()
        t_copy(b1, t_v1, sem_t1).start()

        @pl.when(k > 0)
        def _():
            out_copy(b0 - 2 * CHUNK, out_v0, sem_o0).wait()

        compute_chunk(t_v0, out_v0)
        out_copy(b0, out_v0, sem_o0).start()

        t_copy(b1, t_v1, sem_t1).wait()

        @pl.when(k < N_PAIRS - 1)
        def _():
            t_copy(b0 + 2 * CHUNK, t_v0, sem_t0).start()

        @pl.when(k > 0)
        def _():
            out_copy(b1 - 2 * CHUNK, out_v1, sem_o1).wait()

        compute_chunk(t_v1, out_v1)
        out_copy(b1, out_v1, sem_o1).start()
        return 0

    lax.fori_loop(0, N_PAIRS, pair_body, 0)

    out_copy(my_base, out_v0, sem_o0).wait()
    out_copy(my_base, out_v1, sem_o1).wait()


def kernel(t, I_grid):
    mesh = plsc.VectorSubcoreMesh(core_axis_name="c", subcore_axis_name="s")
    f = pl.kernel(
        _sc_body,
        out_type=jax.ShapeDtypeStruct((N_BASIS, N_POINTS), jnp.float32),
        mesh=mesh,
        compiler_params=pltpu.CompilerParams(needs_layout_passes=False,
                                             use_tc_tiling_on_sc=True),
        scratch_types=[
            pltpu.VMEM((N_GRID * N_BASIS,), jnp.float32),
            pltpu.VMEM((N_GRID * N_BASIS,), jnp.float32),
            pltpu.VMEM((CHUNK + 16,), jnp.float32),
            pltpu.VMEM((CHUNK + 16,), jnp.float32),
            pltpu.VMEM((N_BASIS, CHUNK), jnp.float32),
            pltpu.VMEM((N_BASIS, CHUNK), jnp.float32),
            pltpu.SemaphoreType.DMA,
            pltpu.SemaphoreType.DMA,
            pltpu.SemaphoreType.DMA,
            pltpu.SemaphoreType.DMA,
        ],
    )
    out_t = f(t, I_grid.reshape(-1))
    return out_t.T


# speedup vs baseline: 1.1641x; 1.1641x over previous
import jax
import jax.numpy as jnp
from jax import lax
from jax.experimental import pallas as pl
from jax.experimental.pallas import tpu as pltpu
from jax.experimental.pallas import tpu_sc as plsc

N_POINTS = 819200
N_GRID = 512
N_BASIS = 16

NC = 2
NS = 16
NW = NC * NS

PER_W = N_POINTS // NW
CHUNK = 512
N_CHUNKS = PER_W // CHUNK
N_PAIRS = N_CHUNKS // 2


def _sc_body(t_hbm, grid_hbm, out_hbm, table_v, dtab_v,
             t_v0, t_v1, out_v0, out_v1, sem_t0, sem_t1, sem_o0, sem_o1):
    wid = lax.axis_index("s") * NC + lax.axis_index("c")
    my_base = wid * PER_W

    pltpu.sync_copy(grid_hbm, table_v)
    pltpu.sync_copy(grid_hbm.at[pl.ds(N_BASIS, (N_GRID - 1) * N_BASIS)],
                    dtab_v.at[pl.ds(0, (N_GRID - 1) * N_BASIS)])
    pltpu.sync_copy(grid_hbm.at[pl.ds((N_GRID - 1) * N_BASIS, N_BASIS)],
                    dtab_v.at[pl.ds((N_GRID - 1) * N_BASIS, N_BASIS)])

    lane = lax.iota(jnp.int32, 16)

    def diff_body(r, _):
        off = r * 16
        dtab_v[pl.ds(off, 16)] = (dtab_v[pl.ds(off, 16)]
                                  - table_v[pl.ds(off, 16)])
        return 0

    lax.fori_loop(0, N_GRID, diff_body, 0)

    def prep(t_v, j):
        tvec = t_v[pl.ds(j * 16, 16)]
        u = tvec * jnp.float32(N_GRID - 1)
        i0v = u.astype(jnp.int32)
        wv = u - i0v.astype(jnp.float32)
        return i0v * N_BASIS, wv

    def compute_chunk(t_v, out_v):
        def block_body(j, carry):
            o0v, wv = carry
            carry_n = prep(t_v, j + 1)
            rowv = j * 16 + lane
            for wave in range(0, 8, 4):
                gs = []
                for d in range(wave, wave + 4):
                    dvec = (lane + d) & (N_BASIS - 1)
                    dvec_b = dvec ^ 8
                    idx_a = o0v | dvec
                    idx_b = idx_a ^ 8
                    g0a = plsc.load_gather(table_v, [idx_a])
                    gda = plsc.load_gather(dtab_v, [idx_a])
                    g0b = plsc.load_gather(table_v, [idx_b])
                    gdb = plsc.load_gather(dtab_v, [idx_b])
                    gs.append((dvec, dvec_b, g0a, gda, g0b, gdb))
                for dvec, dvec_b, g0a, gda, g0b, gdb in gs:
                    plsc.store_scatter(out_v, [dvec, rowv], g0a + wv * gda)
                    plsc.store_scatter(out_v, [dvec_b, rowv], g0b + wv * gdb)
            return carry_n

        lax.fori_loop(0, CHUNK // 16, block_body, prep(t_v, 0))

    def t_copy(base, t_v, sem):
        return pltpu.make_async_copy(t_hbm.at[pl.ds(base, CHUNK)],
                                     t_v.at[pl.ds(0, CHUNK)], sem)

    def out_copy(base, out_v, sem):
        return pltpu.make_async_copy(out_v, out_hbm.at[:, pl.ds(base, CHUNK)],
                                     sem)

    t_v0[pl.ds(CHUNK, 16)] = jnp.zeros(16, jnp.float32)
    t_v1[pl.ds(CHUNK, 16)] = jnp.zeros(16, jnp.float32)

    t_copy(my_base, t_v0, sem_t0).start()

    def pair_body(k, _):
        b0 = my_base + (2 * k) * CHUNK
        b1 = b0 + CHUNK

        t_copy(b0, t_v0, sem_t0).wait()
        t_copy(b1, t_v1, sem_t1).start()

        @pl.when(k > 0)
        def _():
            out_copy(b0 - 2 * CHUNK, out_v0, sem_o0).wait()

        compute_chunk(t_v0, out_v0)
        out_copy(b0, out_v0, sem_o0).start()

        t_copy(b1, t_v1, sem_t1).wait()

        @pl.when(k < N_PAIRS - 1)
        def _():
            t_copy(b0 + 2 * CHUNK, t_v0, sem_t0).start()

        @pl.when(k > 0)
        def _():
            out_copy(b1 - 2 * CHUNK, out_v1, sem_o1).wait()

        compute_chunk(t_v1, out_v1)
        out_copy(b1, out_v1, sem_o1).start()
        return 0

    lax.fori_loop(0, N_PAIRS, pair_body, 0)

    out_copy(my_base, out_v0, sem_o0).wait()
    out_copy(my_base, out_v1, sem_o1).wait()


def kernel(t, I_grid):
    mesh = plsc.VectorSubcoreMesh(core_axis_name="c", subcore_axis_name="s")
    f = pl.kernel(
        _sc_body,
        out_type=jax.ShapeDtypeStruct((N_BASIS, N_POINTS), jnp.float32),
        mesh=mesh,
        compiler_params=pltpu.CompilerParams(needs_layout_passes=False,
                                             use_tc_tiling_on_sc=True),
        scratch_types=[
            pltpu.VMEM((N_GRID * N_BASIS,), jnp.float32),
            pltpu.VMEM((N_GRID * N_BASIS,), jnp.float32),
            pltpu.VMEM((CHUNK + 16,), jnp.float32),
            pltpu.VMEM((CHUNK + 16,), jnp.float32),
            pltpu.VMEM((N_BASIS, CHUNK), jnp.float32),
            pltpu.VMEM((N_BASIS, CHUNK), jnp.float32),
            pltpu.SemaphoreType.DMA,
            pltpu.SemaphoreType.DMA,
            pltpu.SemaphoreType.DMA,
            pltpu.SemaphoreType.DMA,
        ],
    )
    out_t = f(t, I_grid.reshape(-1))
    return out_t.T
